# Initial kernel scaffold; baseline (speedup 1.0000x reference)
import jax
import jax.numpy as jnp
from jax.experimental import pallas as pl


def kernel(x, embd):
    raise NotImplementedError("write your pallas kernel here")


# speedup vs baseline: 1.1092x; 1.1092x over previous
import functools

import jax
import jax.numpy as jnp
from jax import lax
from jax.experimental import pallas as pl
from jax.experimental.pallas import tpu as pltpu
from jax.experimental.pallas import tpu_sc as plsc

NUM_CORES = 2
NUM_SUBCORES = 16
NUM_WORKERS = NUM_CORES * NUM_SUBCORES
CHUNK = 3200


@functools.lru_cache(maxsize=None)
def _make_kernel(B: int, D: int):
    b_per_w = B // NUM_WORKERS
    n_chunks = b_per_w // CHUNK
    mesh = plsc.VectorSubcoreMesh(core_axis_name="c", subcore_axis_name="s")

    @functools.partial(
        pl.kernel,
        mesh=mesh,
        out_type=jax.ShapeDtypeStruct((B, D), jnp.float32),
        scratch_types=[
            pltpu.VMEM((CHUNK,), jnp.int32),
            pltpu.VMEM((CHUNK, D), jnp.float32),
            pltpu.SemaphoreType.DMA,
        ],
        compiler_params=pltpu.CompilerParams(use_tc_tiling_on_sc=False),
    )
    def emb_kernel(idx_hbm, table_hbm, out_hbm, idx_v, rows_v, sem):
        wid = lax.axis_index("s") * NUM_CORES + lax.axis_index("c")
        base = wid * b_per_w

        def body(i, carry):
            off = base + i * CHUNK
            pltpu.sync_copy(idx_hbm.at[pl.ds(off, CHUNK)], idx_v)
            pltpu.async_copy(table_hbm.at[idx_v], rows_v, sem).wait()
            pltpu.sync_copy(rows_v, out_hbm.at[pl.ds(off, CHUNK)])
            return carry

        lax.fori_loop(0, n_chunks, body, 0)

    return emb_kernel


def kernel(x, embd):
    B = x.shape[0] * x.shape[1]
    D = embd.shape[1]
    xf = x.reshape(B).astype(jnp.int32)
    out = _make_kernel(B, D)(xf, embd)
    return out.reshape(x.shape[0], x.shape[1], D)


# speedup vs baseline: 1.1126x; 1.0031x over previous
import functools

import jax
import jax.numpy as jnp
from jax import lax
from jax.experimental import pallas as pl
from jax.experimental.pallas import tpu as pltpu
from jax.experimental.pallas import tpu_sc as plsc

NUM_CORES = 2
NUM_SUBCORES = 16
NUM_WORKERS = NUM_CORES * NUM_SUBCORES
NBUF = 2
CHUNK = 1600


@functools.lru_cache(maxsize=None)
def _make_kernel(B: int, D: int):
    b_per_w = B // NUM_WORKERS
    n_chunks = b_per_w // CHUNK
    mesh = plsc.VectorSubcoreMesh(core_axis_name="c", subcore_axis_name="s")

    scratch = (
        [pltpu.VMEM((CHUNK,), jnp.int32) for _ in range(NBUF)]
        + [pltpu.VMEM((CHUNK, D), jnp.float32) for _ in range(NBUF)]
        + [pltpu.SemaphoreType.DMA] * (3 * NBUF)
    )

    @functools.partial(
        pl.kernel,
        mesh=mesh,
        out_type=jax.ShapeDtypeStruct((B, D), jnp.float32),
        scratch_types=scratch,
        compiler_params=pltpu.CompilerParams(use_tc_tiling_on_sc=False),
    )
    def emb_kernel(idx_hbm, table_hbm, out_hbm, *refs):
        idx_bufs = refs[0:NBUF]
        row_bufs = refs[NBUF:2 * NBUF]
        sem_i = refs[2 * NBUF:3 * NBUF]
        sem_g = refs[3 * NBUF:4 * NBUF]
        sem_s = refs[4 * NBUF:5 * NBUF]

        wid = lax.axis_index("s") * NUM_CORES + lax.axis_index("c")
        base = wid * b_per_w

        idx_cp = [None] * NBUF
        gat_cp = [None] * NBUF
        st_cp = [None] * NBUF

        def start_idx(i):
            b = i % NBUF
            idx_cp[b] = pltpu.async_copy(
                idx_hbm.at[pl.ds(base + i * CHUNK, CHUNK)], idx_bufs[b], sem_i[b])

        def start_gather(i):
            b = i % NBUF
            idx_cp[b].wait()
            gat_cp[b] = pltpu.async_copy(
                table_hbm.at[idx_bufs[b]], row_bufs[b], sem_g[b])

        def start_store(i):
            b = i % NBUF
            st_cp[b] = pltpu.async_copy(
                row_bufs[b], out_hbm.at[pl.ds(base + i * CHUNK, CHUNK)], sem_s[b])

        for k in range(min(NBUF, n_chunks)):
            start_idx(k)
        start_gather(0)

        for i in range(n_chunks):
            b = i % NBUF
            if i + 1 < n_chunks:
                if i + 1 >= NBUF:
                    st_cp[(i + 1) % NBUF].wait()
                start_gather(i + 1)
            gat_cp[b].wait()
            start_store(i)
            if i + NBUF < n_chunks:
                start_idx(i + NBUF)

        for i in range(max(0, n_chunks - NBUF), n_chunks):
            st_cp[i % NBUF].wait()

    return emb_kernel


def kernel(x, embd):
    B = x.shape[0] * x.shape[1]
    D = embd.shape[1]
    xf = x.reshape(B).astype(jnp.int32)
    out = _make_kernel(B, D)(xf, embd)
    return out.reshape(x.shape[0], x.shape[1], D)


# speedup vs baseline: 1.3858x; 1.2456x over previous
import functools

import jax
import jax.numpy as jnp
from jax import lax
from jax.experimental import pallas as pl
from jax.experimental.pallas import tpu as pltpu
from jax.experimental.pallas import tpu_sc as plsc

NUM_CORES = 2
NUM_SUBCORES = 16
NUM_WORKERS = NUM_CORES * NUM_SUBCORES
BLK = 128


@functools.lru_cache(maxsize=None)
def _make_kernel(NI: int, NJ: int, V: int, D: int):
    n_blocks = NI // BLK * NJ
    per_w = n_blocks // NUM_WORKERS
    blocks_per_j = NI // BLK
    mesh = plsc.VectorSubcoreMesh(core_axis_name="c", subcore_axis_name="s")

    scratch = (
        [pltpu.VMEM((BLK,), jnp.int32) for _ in range(2)]
        + [pltpu.VMEM((BLK, D), jnp.float32) for _ in range(2)]
        + [pltpu.VMEM((D, BLK), jnp.float32) for _ in range(2)]
        + [pltpu.SemaphoreType.DMA] * 6
    )

    @functools.partial(
        pl.kernel,
        mesh=mesh,
        out_type=jax.ShapeDtypeStruct((NJ, D, NI), jnp.float32),
        scratch_types=scratch,
        compiler_params=pltpu.CompilerParams(
            use_tc_tiling_on_sc=False, needs_layout_passes=False),
    )
    def emb_kernel(xT_hbm, table_hbm, out_hbm, i0, i1, r0, r1, t0, t1,
                   si0, si1, sg0, sg1, ss0, ss1):
        idx_b = (i0, i1)
        rows_b = (r0, r1)
        rowsT_b = (t0, t1)
        si = (si0, si1)
        sg = (sg0, sg1)
        ss = (ss0, ss1)

        w = lax.axis_index("s") * NUM_CORES + lax.axis_index("c")
        base = w * per_w

        row_ids = [lax.iota(jnp.int32, 16) + 16 * k for k in range(8)]

        def idx_slice(g):
            j = g // blocks_per_j
            b = g % blocks_per_j
            return xT_hbm.at[j, pl.ds(b * BLK, BLK)]

        def out_slice(g):
            j = g // blocks_per_j
            b = g % blocks_per_j
            return out_hbm.at[j, :, pl.ds(b * BLK, BLK)]

        def idx_cp(g, p):
            return pltpu.make_async_copy(idx_slice(g), idx_b[p], si[p])

        def gather_cp(p):
            return pltpu.make_async_copy(table_hbm.at[idx_b[p]], rows_b[p], sg[p])

        def store_cp(g, p):
            return pltpu.make_async_copy(rowsT_b[p], out_slice(g), ss[p])

        idx_cp(base + 0, 0).start()
        idx_cp(base + 1, 1).start()
        idx_cp(base + 0, 0).wait()
        gather_cp(0).start()

        def body(i, carry):
            for p in (0, 1):
                blk = 2 * i + p
                g = base + blk
                p1 = p ^ 1

                @pl.when(blk + 1 <= per_w - 1)
                def _():
                    idx_cp(g + 1, p1).wait()
                    gather_cp(p1).start()

                gather_cp(p).wait()

                @pl.when(blk + 2 <= per_w - 1)
                def _():
                    idx_cp(g + 2, p).start()

                @pl.when(blk >= 2)
                def _():
                    store_cp(g - 2, p).wait()

                for d in range(D):
                    col_d = jnp.full((16,), d, jnp.int32)
                    for k in range(8):
                        val = plsc.load_gather(rows_b[p], [row_ids[k], col_d])
                        rowsT_b[p][d, pl.ds(16 * k, 16)] = val

                store_cp(g, p).start()
            return carry

        lax.fori_loop(0, per_w // 2, body, 0)
        store_cp(base + per_w - 2, 0).wait()
        store_cp(base + per_w - 1, 1).wait()

    return emb_kernel


def kernel(x, embd):
    NI, NJ = x.shape
    V, D = embd.shape
    xT = x.T.astype(jnp.int32)
    outT = _make_kernel(NI, NJ, V, D)(xT, embd)
    return jnp.transpose(outT, (2, 0, 1))


# speedup vs baseline: 1.8031x; 1.3011x over previous
import functools

import jax
import jax.numpy as jnp
from jax import lax
from jax.experimental import pallas as pl
from jax.experimental.pallas import tpu as pltpu
from jax.experimental.pallas import tpu_sc as plsc

NUM_CORES = 2
NUM_SUBCORES = 16
NUM_WORKERS = NUM_CORES * NUM_SUBCORES
NBUF = 2
ROWS_PER_CHUNK = 32


@functools.lru_cache(maxsize=None)
def _make_kernel(NI: int, NJ: int, D: int):
    B = NI * NJ
    CHUNK = ROWS_PER_CHUNK * NJ
    b_per_w = B // NUM_WORKERS
    n_chunks = b_per_w // CHUNK
    rows_per_w = NI // NUM_WORKERS
    mesh = plsc.VectorSubcoreMesh(core_axis_name="c", subcore_axis_name="s")

    scratch = (
        [pltpu.VMEM((CHUNK,), jnp.int32) for _ in range(NBUF)]
        + [pltpu.VMEM((CHUNK, D), jnp.float32) for _ in range(NBUF)]
        + [pltpu.SemaphoreType.DMA] * (3 * NBUF)
    )

    @functools.partial(
        pl.kernel,
        mesh=mesh,
        out_type=jax.ShapeDtypeStruct((NI, NJ, D), jnp.float32),
        scratch_types=scratch,
        compiler_params=pltpu.CompilerParams(
            use_tc_tiling_on_sc=False, needs_layout_passes=False),
    )
    def emb_kernel(idx_hbm, table_hbm, out_hbm, *refs):
        idx_bufs = refs[0:NBUF]
        row_bufs = refs[NBUF:2 * NBUF]
        sem_i = refs[2 * NBUF:3 * NBUF]
        sem_g = refs[3 * NBUF:4 * NBUF]
        sem_s = refs[4 * NBUF:5 * NBUF]

        wid = lax.axis_index("s") * NUM_CORES + lax.axis_index("c")
        base = wid * b_per_w
        i_base = wid * rows_per_w

        idx_cp = [None] * NBUF
        gat_cp = [None] * NBUF
        st_cp = [[None] * ROWS_PER_CHUNK for _ in range(NBUF)]

        def start_idx(i):
            b = i % NBUF
            idx_cp[b] = pltpu.async_copy(
                idx_hbm.at[pl.ds(base + i * CHUNK, CHUNK)], idx_bufs[b], sem_i[b])

        def start_gather(i):
            b = i % NBUF
            idx_cp[b].wait()
            gat_cp[b] = pltpu.async_copy(
                table_hbm.at[idx_bufs[b]], row_bufs[b], sem_g[b])

        def start_store(i):
            b = i % NBUF
            for r in range(ROWS_PER_CHUNK):
                st_cp[b][r] = pltpu.async_copy(
                    row_bufs[b].at[pl.ds(r * NJ, NJ), :],
                    out_hbm.at[i_base + i * ROWS_PER_CHUNK + r],
                    sem_s[b])

        def wait_store(i):
            b = i % NBUF
            for r in range(ROWS_PER_CHUNK):
                st_cp[b][r].wait()

        for k in range(min(NBUF, n_chunks)):
            start_idx(k)
        start_gather(0)

        for i in range(n_chunks):
            b = i % NBUF
            if i + 1 < n_chunks:
                if i + 1 >= NBUF:
                    wait_store(i + 1 - NBUF)
                start_gather(i + 1)
            gat_cp[b].wait()
            start_store(i)
            if i + NBUF < n_chunks:
                start_idx(i + NBUF)

        for i in range(max(0, n_chunks - NBUF), n_chunks):
            wait_store(i)

    return emb_kernel


def kernel(x, embd):
    NI, NJ = x.shape
    D = embd.shape[1]
    xf = x.reshape(NI * NJ).astype(jnp.int32)
    return _make_kernel(NI, NJ, D)(xf, embd)


# speedup vs baseline: 1.8410x; 1.0210x over previous
import functools

import jax
import jax.numpy as jnp
from jax import lax
from jax.experimental import pallas as pl
from jax.experimental.pallas import tpu as pltpu
from jax.experimental.pallas import tpu_sc as plsc

NUM_CORES = 2
NUM_SUBCORES = 16
NUM_WORKERS = NUM_CORES * NUM_SUBCORES
BLK = 128


@functools.lru_cache(maxsize=None)
def _make_kernel(NI: int, NJ: int, D: int):
    blocks_per_j = NI // BLK
    n_blocks = blocks_per_j * NJ
    per_w = n_blocks // NUM_WORKERS
    mesh = plsc.VectorSubcoreMesh(core_axis_name="c", subcore_axis_name="s")

    scratch = (
        [pltpu.VMEM((BLK,), jnp.int32) for _ in range(2)]
        + [pltpu.VMEM((BLK, D), jnp.float32) for _ in range(2)]
        + [pltpu.VMEM((D // 8, 1, 8, BLK), jnp.float32) for _ in range(2)]
        + [pltpu.SemaphoreType.DMA] * 6
    )

    @functools.partial(
        pl.kernel,
        mesh=mesh,
        out_type=jax.ShapeDtypeStruct((NJ, D // 8, blocks_per_j, 8, BLK),
                                      jnp.float32),
        scratch_types=scratch,
        compiler_params=pltpu.CompilerParams(
            use_tc_tiling_on_sc=False, needs_layout_passes=False),
    )
    def emb_kernel(xT_hbm, table_hbm, out_hbm, i0, i1, r0, r1, t0, t1,
                   si0, si1, sg0, sg1, ss0, ss1):
        idx_b = (i0, i1)
        rows_b = (r0, r1)
        rowsT_b = (t0, t1)
        si = (si0, si1)
        sg = (sg0, sg1)
        ss = (ss0, ss1)

        w = lax.axis_index("s") * NUM_CORES + lax.axis_index("c")
        base = w * per_w

        dim_sel = (lax.iota(jnp.int32, 16), lax.iota(jnp.int32, 16) + 16)
        a_sel = tuple(s // 8 for s in dim_sel)
        dd_sel = tuple(s % 8 for s in dim_sel)
        z_sel = jnp.zeros((16,), jnp.int32)

        def idx_cp(g, p):
            j = g // blocks_per_j
            b = g % blocks_per_j
            return pltpu.make_async_copy(
                xT_hbm.at[j, pl.ds(b * BLK, BLK)], idx_b[p], si[p])

        def gather_cp(p):
            return pltpu.make_async_copy(table_hbm.at[idx_b[p]], rows_b[p], sg[p])

        def store_cp(g, p):
            j = g // blocks_per_j
            b = g % blocks_per_j
            return pltpu.make_async_copy(
                rowsT_b[p], out_hbm.at[j, :, pl.ds(b, 1), :, :], ss[p])

        idx_cp(base + 0, 0).start()
        idx_cp(base + 1, 1).start()
        idx_cp(base + 0, 0).wait()
        gather_cp(0).start()

        def body(i, carry):
            for p in (0, 1):
                blk = 2 * i + p
                g = base + blk
                p1 = p ^ 1

                @pl.when(blk + 1 <= per_w - 1)
                def _():
                    idx_cp(g + 1, p1).wait()
                    gather_cp(p1).start()

                gather_cp(p).wait()

                @pl.when(blk + 2 <= per_w - 1)
                def _():
                    idx_cp(g + 2, p).start()

                @pl.when(blk >= 2)
                def _():
                    store_cp(g - 2, p).wait()

                for r in range(BLK):
                    col_r = jnp.full((16,), r, jnp.int32)
                    for h in (0, 1):
                        v = rows_b[p][r, pl.ds(16 * h, 16)]
                        plsc.store_scatter(
                            rowsT_b[p], [a_sel[h], z_sel, dd_sel[h], col_r], v)

                store_cp(g, p).start()
            return carry

        lax.fori_loop(0, per_w // 2, body, 0)
        store_cp(base + per_w - 2, 0).wait()
        store_cp(base + per_w - 1, 1).wait()

    return emb_kernel


def kernel(x, embd):
    NI, NJ = x.shape
    D = embd.shape[1]
    xT = x.T.astype(jnp.int32)
    outP = _make_kernel(NI, NJ, D)(xT, embd)
    out = jnp.transpose(outP, (2, 4, 0, 1, 3)).reshape(NI, NJ, D)
    return out


# speedup vs baseline: 2.1522x; 1.1690x over previous
import functools

import jax
import jax.numpy as jnp
from jax import lax
from jax.experimental import pallas as pl
from jax.experimental.pallas import tpu as pltpu
from jax.experimental.pallas import tpu_sc as plsc

NUM_CORES = 2
NUM_SUBCORES = 16
NUM_WORKERS = NUM_CORES * NUM_SUBCORES
BLK = 128


@functools.lru_cache(maxsize=None)
def _make_kernel(NI: int, NJ: int, D: int):
    blocks_per_j = NI // BLK
    n_blocks = blocks_per_j * NJ
    per_w = n_blocks // NUM_WORKERS
    mesh = plsc.VectorSubcoreMesh(core_axis_name="c", subcore_axis_name="s")

    scratch = (
        [pltpu.VMEM((BLK,), jnp.int32) for _ in range(2)]
        + [pltpu.VMEM((BLK, D), jnp.float32) for _ in range(2)]
        + [pltpu.VMEM((D // 8, 1, 8, BLK), jnp.float32) for _ in range(2)]
        + [pltpu.SemaphoreType.DMA] * 6
    )

    @functools.partial(
        pl.kernel,
        mesh=mesh,
        out_type=jax.ShapeDtypeStruct((NJ, D // 8, blocks_per_j, 8, BLK),
                                      jnp.float32),
        scratch_types=scratch,
        compiler_params=pltpu.CompilerParams(
            use_tc_tiling_on_sc=False, needs_layout_passes=False),
    )
    def emb_kernel(xT_hbm, table_hbm, out_hbm, i0, i1, r0, r1, t0, t1,
                   si0, si1, sg0, sg1, ss0, ss1):
        idx_b = (i0, i1)
        rows_b = (r0, r1)
        rowsT_b = (t0, t1)
        si = (si0, si1)
        sg = (sg0, sg1)
        ss = (ss0, ss1)

        w = lax.axis_index("s") * NUM_CORES + lax.axis_index("c")
        base = w * per_w

        dim_sel = (lax.iota(jnp.int32, 16), lax.iota(jnp.int32, 16) + 16)
        a_sel = tuple(s // 8 for s in dim_sel)
        dd_sel = tuple(s % 8 for s in dim_sel)
        z_sel = jnp.zeros((16,), jnp.int32)

        def idx_cp(g, p):
            j = g // blocks_per_j
            b = g % blocks_per_j
            return pltpu.make_async_copy(
                xT_hbm.at[j, pl.ds(b * BLK, BLK)], idx_b[p], si[p])

        def gather_cp(p):
            return pltpu.make_async_copy(table_hbm.at[idx_b[p]], rows_b[p], sg[p])

        def store_cp(g, p):
            j = g // blocks_per_j
            b = g % blocks_per_j
            return pltpu.make_async_copy(
                rowsT_b[p], out_hbm.at[j, :, pl.ds(b, 1), :, :], ss[p])

        idx_cp(base + 0, 0).start()
        idx_cp(base + 1, 1).start()
        idx_cp(base + 0, 0).wait()
        gather_cp(0).start()

        def body(i, carry):
            for p in (0, 1):
                blk = 2 * i + p
                g = base + blk
                p1 = p ^ 1

                @pl.when(blk + 1 <= per_w - 1)
                def _():
                    idx_cp(g + 1, p1).wait()
                    gather_cp(p1).start()

                gather_cp(p).wait()

                @pl.when(blk + 2 <= per_w - 1)
                def _():
                    idx_cp(g + 2, p).start()

                @pl.when(blk >= 2)
                def _():
                    store_cp(g - 2, p).wait()

                @plsc.parallel_loop(0, BLK, 1, unroll=8)
                def _(r):
                    col_r = jnp.full((16,), r, jnp.int32)
                    for h in (0, 1):
                        v = rows_b[p][r, pl.ds(16 * h, 16)]
                        plsc.store_scatter(
                            rowsT_b[p], [a_sel[h], z_sel, dd_sel[h], col_r], v)

                store_cp(g, p).start()
            return carry

        lax.fori_loop(0, per_w // 2, body, 0)
        store_cp(base + per_w - 2, 0).wait()
        store_cp(base + per_w - 1, 1).wait()

    return emb_kernel


def kernel(x, embd):
    NI, NJ = x.shape
    D = embd.shape[1]
    xT = x.T.astype(jnp.int32)
    outP = _make_kernel(NI, NJ, D)(xT, embd)
    out = jnp.transpose(outP, (2, 4, 0, 1, 3)).reshape(NI, NJ, D)
    return out
